# baseline (speedup 1.0000x reference)
import functools

import numpy as np

import jax
import jax.numpy as jnp
from jax import lax
from jax.experimental import pallas as pl
from jax.experimental.pallas import tpu as pltpu
from jax.experimental.pallas import tpu_sc as plsc

N_NODES = 10000
N_EDGES = 320000
D_IN = 128
H1 = 128
H2 = 64

NC = 2
NS = 16
NW = NC * NS
CHUNK = 128
E_PAD = 327680
NCHT = E_PAD // NS // CHUNK
NCHW = E_PAD // NW // CHUNK
NBUF = 4
NPAD = 10112
RPT = NPAD // NS
DEGR = 10240

BM = 10240
GRID = 1

_PI = np.arange(E_PAD - N_EDGES, dtype=np.int32)
_PAD_SRC = _PI & 4095
_PAD_DST = (N_NODES + _PI % (NPAD - N_NODES)).astype(np.int32)

_mesh = plsc.VectorSubcoreMesh(core_axis_name="c", subcore_axis_name="s")


@functools.partial(
    pl.kernel,
    mesh=_mesh,
    out_type=jax.ShapeDtypeStruct((NW, DEGR), jnp.float32),
    compiler_params=pltpu.CompilerParams(needs_layout_passes=False),
    scratch_types=[
        pltpu.VMEM((NCHW, CHUNK), jnp.int32),
        pltpu.VMEM((DEGR,), jnp.float32),
    ],
)
def _sc_deg(dst_hbm, out_hbm, didx2, hist):
    cid = lax.axis_index("c")
    sid = lax.axis_index("s")
    wid = cid * NS + sid
    zero16 = jnp.zeros((16,), jnp.float32)
    one16 = jnp.full((16,), 1.0, jnp.float32)

    def z(i, _):
        hist[pl.ds(i * 16, 16)] = zero16
        return 0
    lax.fori_loop(0, DEGR // 16, z, 0)

    pltpu.sync_copy(dst_hbm.at[sid].at[pl.ds(cid * NCHW, NCHW)], didx2)

    def body(i, _):
        r = i // (CHUNK // 16)
        c = i % (CHUNK // 16)
        q = didx2[r, pl.ds(c * 16, 16)]
        plsc.addupdate_scatter(hist, [q], one16)
        return 0
    lax.fori_loop(0, (NCHW * CHUNK) // 16, body, 0)

    pltpu.sync_copy(hist, out_hbm.at[wid])


def _build_scatter(d_feat):
    half = d_feat // 2

    @functools.partial(
        pl.kernel,
        mesh=_mesh,
        out_type=jax.ShapeDtypeStruct((NC, NPAD, half), jnp.float32),
        compiler_params=pltpu.CompilerParams(use_tc_tiling_on_sc=False),
        scratch_types=(
            [pltpu.VMEM((NCHT, CHUNK), jnp.int32)] * 2
            + [pltpu.VMEM((CHUNK, half), jnp.float32)] * NBUF
            + [pltpu.VMEM_SHARED((NPAD, half), jnp.float32)]
            + [pltpu.SemaphoreType.DMA] * (2 * NBUF)
        ),
    )
    def scat(src_hbm, dst_hbm, tab_hbm, out_hbm, sidx2, didx2,
             b0, b1, b2, b3, acc, g0, g1, g2, g3, s0, s1, s2, s3):
        cid = lax.axis_index("c")
        sid = lax.axis_index("s")
        bufs = (b0, b1, b2, b3)
        gsem = (g0, g1, g2, g3)
        ssem = (s0, s1, s2, s3)
        zero16 = jnp.zeros((16,), jnp.float32)

        def zrow(r, _):
            for c in range(half // 16):
                b0[r, pl.ds(c * 16, 16)] = zero16
            return 0
        lax.fori_loop(0, CHUNK, zrow, 0)

        base = sid * RPT
        for k in range(RPT // CHUNK):
            pltpu.sync_copy(b0, acc.at[pl.ds(base + k * CHUNK, CHUNK)])
        rem = RPT % CHUNK
        if rem:
            pltpu.sync_copy(b0.at[pl.ds(0, rem)],
                            acc.at[pl.ds(base + (RPT // CHUNK) * CHUNK, rem)])

        pltpu.sync_copy(src_hbm.at[sid], sidx2)
        pltpu.sync_copy(dst_hbm.at[sid], didx2)
        plsc.subcore_barrier()

        tabc = tab_hbm.at[cid]

        def gather(j, b):
            pltpu.async_copy(tabc.at[sidx2.at[j]], bufs[b], gsem[b])

        def gwait(j, b):
            pltpu.make_async_copy(tabc.at[sidx2.at[j]], bufs[b],
                                  gsem[b]).wait()

        def scatter(j, b):
            pltpu.async_copy(bufs[b], acc.at[didx2.at[j]], ssem[b], add=True)

        def swait(j, b):
            pltpu.make_async_copy(bufs[b], acc.at[didx2.at[j]],
                                  ssem[b]).wait()

        for b in range(NBUF):
            gather(b, b)

        def body(g, _):
            j0 = g * NBUF
            for b in range(NBUF):
                gwait(j0 + b, b)
                scatter(j0 + b, b)
            for b in range(NBUF):
                swait(j0 + b, b)
                gather(j0 + NBUF + b, b)
            return 0
        lax.fori_loop(0, NCHT // NBUF - 1, body, 0)

        j0 = NCHT - NBUF
        for b in range(NBUF):
            gwait(j0 + b, b)
            scatter(j0 + b, b)
        for b in range(NBUF):
            swait(j0 + b, b)

        plsc.subcore_barrier()
        pltpu.sync_copy(acc.at[pl.ds(sid * RPT, RPT)],
                        out_hbm.at[cid].at[pl.ds(sid * RPT, RPT)])

    return scat


_scatter_h1 = _build_scatter(H1)
_scatter_h2 = _build_scatter(H2)


def _mm1_body(x_ref, w_ref, degp_ref, o_ref, dinv_ref):
    i = pl.program_id(0)
    deg = 1.0 + jnp.sum(degp_ref[:, pl.ds(i * BM, BM)], axis=0)
    dinv = lax.rsqrt(deg)[:, None]
    dinv_ref[...] = dinv
    h = jnp.dot(x_ref[...], w_ref[...], preferred_element_type=jnp.float32)
    h = h * dinv
    o_ref[0] = h[:, :H1 // 2]
    o_ref[1] = h[:, H1 // 2:]


def _mid_body(h1p_ref, p_ref, dinv_ref, b1_ref, w2_ref, o_ref):
    dinv = dinv_ref[...]
    t = (jnp.concatenate([p_ref[0], p_ref[1]], axis=1)
         + jnp.concatenate([h1p_ref[0], h1p_ref[1]], axis=1))
    t = jnp.maximum(t * dinv + b1_ref[0], 0.0)
    h2 = jnp.dot(t, w2_ref[...], preferred_element_type=jnp.float32) * dinv
    o_ref[0] = h2[:, :H2 // 2]
    o_ref[1] = h2[:, H2 // 2:]


def _out_body(h2p_ref, q_ref, dinv_ref, b2_ref, o_ref):
    dinv = dinv_ref[...]
    t = (jnp.concatenate([q_ref[0], q_ref[1]], axis=1)
         + jnp.concatenate([h2p_ref[0], h2p_ref[1]], axis=1))
    o_ref[...] = jnp.maximum(t * dinv + b2_ref[0], 0.0)


def _dinv_spec():
    return pl.BlockSpec((BM, 1), lambda i: (i, 0))


def kernel(x, edge_index, W1, b1, W2, b2):
    src = edge_index[0]
    dst = edge_index[1]

    srcT = jnp.concatenate(
        [src, jnp.asarray(_PAD_SRC)]).reshape(NS, NCHT, CHUNK)
    dstT = jnp.concatenate(
        [dst, jnp.asarray(_PAD_DST)]).reshape(NS, NCHT, CHUNK)

    degp = _sc_deg(dstT)

    h1p, dinv = pl.pallas_call(
        _mm1_body,
        grid=(GRID,),
        in_specs=[
            pl.BlockSpec((BM, D_IN), lambda i: (i, 0)),
            pl.BlockSpec((D_IN, H1), lambda i: (0, 0)),
            pl.BlockSpec((NW, DEGR), lambda i: (0, 0)),
        ],
        out_specs=[
            pl.BlockSpec((NC, BM, H1 // 2), lambda i: (0, i, 0)),
            _dinv_spec(),
        ],
        out_shape=[
            jax.ShapeDtypeStruct((NC, N_NODES, H1 // 2), jnp.float32),
            jax.ShapeDtypeStruct((N_NODES, 1), jnp.float32),
        ],
    )(x, W1, degp)

    p1 = _scatter_h1(srcT, dstT, h1p)

    h2p = pl.pallas_call(
        _mid_body,
        grid=(GRID,),
        in_specs=[
            pl.BlockSpec((NC, BM, H1 // 2), lambda i: (0, i, 0)),
            pl.BlockSpec((NC, BM, H1 // 2), lambda i: (0, i, 0)),
            _dinv_spec(),
            pl.BlockSpec((1, H1), lambda i: (0, 0)),
            pl.BlockSpec((H1, H2), lambda i: (0, 0)),
        ],
        out_specs=pl.BlockSpec((NC, BM, H2 // 2), lambda i: (0, i, 0)),
        out_shape=jax.ShapeDtypeStruct((NC, N_NODES, H2 // 2), jnp.float32),
    )(h1p, p1, dinv, b1.reshape(1, H1), W2)

    p2 = _scatter_h2(srcT, dstT, h2p)

    out = pl.pallas_call(
        _out_body,
        grid=(GRID,),
        in_specs=[
            pl.BlockSpec((NC, BM, H2 // 2), lambda i: (0, i, 0)),
            pl.BlockSpec((NC, BM, H2 // 2), lambda i: (0, i, 0)),
            _dinv_spec(),
            pl.BlockSpec((1, H2), lambda i: (0, 0)),
        ],
        out_specs=pl.BlockSpec((BM, H2), lambda i: (i, 0)),
        out_shape=jax.ShapeDtypeStruct((N_NODES, H2), jnp.float32),
    )(h2p, p2, dinv, b2.reshape(1, H2))

    return out


# speedup vs baseline: 1.0239x; 1.0239x over previous
import functools

import numpy as np

import jax
import jax.numpy as jnp
from jax import lax
from jax.experimental import pallas as pl
from jax.experimental.pallas import tpu as pltpu
from jax.experimental.pallas import tpu_sc as plsc

N_NODES = 10000
N_EDGES = 320000
D_IN = 128
H1 = 128
H2 = 64

NC = 2
NS = 16
NW = NC * NS
CHUNK = 128
E_PAD = 327680
NCHT = E_PAD // NS // CHUNK
NCHW = E_PAD // NW // CHUNK
NBUF = 4
NPAD = 10112
RPT = NPAD // NS
DEGR = 10240

BM = 10240
GRID = 1

_PI = np.arange(E_PAD - N_EDGES, dtype=np.int32)
_PAD_SRC = _PI & 4095
_PAD_DST = (N_NODES + _PI % (NPAD - N_NODES)).astype(np.int32)

_mesh = plsc.VectorSubcoreMesh(core_axis_name="c", subcore_axis_name="s")


@functools.partial(
    pl.kernel,
    mesh=_mesh,
    out_type=jax.ShapeDtypeStruct((NW, DEGR), jnp.float32),
    compiler_params=pltpu.CompilerParams(needs_layout_passes=False),
    scratch_types=[
        pltpu.VMEM((NCHW, CHUNK), jnp.int32),
        pltpu.VMEM((DEGR,), jnp.float32),
        pltpu.SemaphoreType.DMA,
    ],
)
def _sc_deg(dst_hbm, out_hbm, didx2, hist, isem):
    cid = lax.axis_index("c")
    sid = lax.axis_index("s")
    wid = cid * NS + sid
    zero16 = jnp.zeros((16,), jnp.float32)
    one16 = jnp.full((16,), 1.0, jnp.float32)

    dsrc = dst_hbm.at[sid].at[pl.ds(cid * NCHW, NCHW)]
    pltpu.async_copy(dsrc, didx2, isem)

    def z(i, _):
        hist[pl.ds(i * 16, 16)] = zero16
        return 0
    lax.fori_loop(0, DEGR // 16, z, 0)

    pltpu.make_async_copy(dsrc, didx2, isem).wait()

    def body(i, _):
        r = i // (CHUNK // 16)
        c = i % (CHUNK // 16)
        q = didx2[r, pl.ds(c * 16, 16)]
        plsc.addupdate_scatter(hist, [q], one16)
        return 0
    lax.fori_loop(0, (NCHW * CHUNK) // 16, body, 0)

    pltpu.sync_copy(hist, out_hbm.at[wid])


def _build_scatter(d_feat):
    half = d_feat // 2

    @functools.partial(
        pl.kernel,
        mesh=_mesh,
        out_type=jax.ShapeDtypeStruct((NC, NPAD, half), jnp.float32),
        compiler_params=pltpu.CompilerParams(use_tc_tiling_on_sc=False),
        scratch_types=(
            [pltpu.VMEM((NCHT, CHUNK), jnp.int32)] * 2
            + [pltpu.VMEM((CHUNK, half), jnp.float32)] * NBUF
            + [pltpu.VMEM_SHARED((NPAD, half), jnp.float32)]
            + [pltpu.SemaphoreType.DMA] * (2 * NBUF + 2)
        ),
    )
    def scat(src_hbm, dst_hbm, tab_hbm, out_hbm, sidx2, didx2,
             b0, b1, b2, b3, acc, g0, g1, g2, g3, s0, s1, s2, s3, i0, i1):
        cid = lax.axis_index("c")
        sid = lax.axis_index("s")
        bufs = (b0, b1, b2, b3)
        gsem = (g0, g1, g2, g3)
        ssem = (s0, s1, s2, s3)
        zero16 = jnp.zeros((16,), jnp.float32)

        pltpu.async_copy(src_hbm.at[sid], sidx2, i0)
        pltpu.async_copy(dst_hbm.at[sid], didx2, i1)

        def zrow(r, _):
            for c in range(half // 16):
                b0[r, pl.ds(c * 16, 16)] = zero16
            return 0
        lax.fori_loop(0, CHUNK, zrow, 0)

        base = sid * RPT
        for k in range(RPT // CHUNK):
            pltpu.sync_copy(b0, acc.at[pl.ds(base + k * CHUNK, CHUNK)])
        rem = RPT % CHUNK
        if rem:
            pltpu.sync_copy(b0.at[pl.ds(0, rem)],
                            acc.at[pl.ds(base + (RPT // CHUNK) * CHUNK, rem)])

        pltpu.make_async_copy(src_hbm.at[sid], sidx2, i0).wait()
        pltpu.make_async_copy(dst_hbm.at[sid], didx2, i1).wait()
        plsc.subcore_barrier()

        tabc = tab_hbm.at[cid]

        def gather(j, b):
            pltpu.async_copy(tabc.at[sidx2.at[j]], bufs[b], gsem[b])

        def gwait(j, b):
            pltpu.make_async_copy(tabc.at[sidx2.at[j]], bufs[b],
                                  gsem[b]).wait()

        def scatter(j, b):
            pltpu.async_copy(bufs[b], acc.at[didx2.at[j]], ssem[b], add=True)

        def swait(j, b):
            pltpu.make_async_copy(bufs[b], acc.at[didx2.at[j]],
                                  ssem[b]).wait()

        for b in range(NBUF):
            gather(b, b)

        def body(g, _):
            j0 = g * NBUF
            for b in range(NBUF):
                gwait(j0 + b, b)
                scatter(j0 + b, b)
            for b in range(NBUF):
                swait(j0 + b, b)
                gather(j0 + NBUF + b, b)
            return 0
        lax.fori_loop(0, NCHT // NBUF - 1, body, 0)

        j0 = NCHT - NBUF
        for b in range(NBUF):
            gwait(j0 + b, b)
            scatter(j0 + b, b)
        for b in range(NBUF):
            swait(j0 + b, b)

        plsc.subcore_barrier()
        pltpu.sync_copy(acc.at[pl.ds(sid * RPT, RPT)],
                        out_hbm.at[cid].at[pl.ds(sid * RPT, RPT)])

    return scat


_scatter_h1 = _build_scatter(H1)
_scatter_h2 = _build_scatter(H2)


def _mm1_body(x_ref, w_ref, degp_ref, o_ref, dinv_ref):
    i = pl.program_id(0)
    deg = 1.0 + jnp.sum(degp_ref[:, pl.ds(i * BM, BM)], axis=0)
    dinv = lax.rsqrt(deg)[:, None]
    dinv_ref[...] = dinv
    h = jnp.dot(x_ref[...], w_ref[...], preferred_element_type=jnp.float32)
    h = h * dinv
    o_ref[0] = h[:, :H1 // 2]
    o_ref[1] = h[:, H1 // 2:]


def _mid_body(h1p_ref, p_ref, dinv_ref, b1_ref, w2_ref, o_ref):
    dinv = dinv_ref[...]
    t = (jnp.concatenate([p_ref[0], p_ref[1]], axis=1)
         + jnp.concatenate([h1p_ref[0], h1p_ref[1]], axis=1))
    t = jnp.maximum(t * dinv + b1_ref[0], 0.0)
    h2 = jnp.dot(t, w2_ref[...], preferred_element_type=jnp.float32) * dinv
    o_ref[0] = h2[:, :H2 // 2]
    o_ref[1] = h2[:, H2 // 2:]


def _out_body(h2p_ref, q_ref, dinv_ref, b2_ref, o_ref):
    dinv = dinv_ref[...]
    t = (jnp.concatenate([q_ref[0], q_ref[1]], axis=1)
         + jnp.concatenate([h2p_ref[0], h2p_ref[1]], axis=1))
    o_ref[...] = jnp.maximum(t * dinv + b2_ref[0], 0.0)


def _dinv_spec():
    return pl.BlockSpec((BM, 1), lambda i: (i, 0))


def kernel(x, edge_index, W1, b1, W2, b2):
    src = edge_index[0]
    dst = edge_index[1]

    srcT = jnp.concatenate(
        [src, jnp.asarray(_PAD_SRC)]).reshape(NS, NCHT, CHUNK)
    dstT = jnp.concatenate(
        [dst, jnp.asarray(_PAD_DST)]).reshape(NS, NCHT, CHUNK)

    degp = _sc_deg(dstT)

    h1p, dinv = pl.pallas_call(
        _mm1_body,
        grid=(GRID,),
        in_specs=[
            pl.BlockSpec((BM, D_IN), lambda i: (i, 0)),
            pl.BlockSpec((D_IN, H1), lambda i: (0, 0)),
            pl.BlockSpec((NW, DEGR), lambda i: (0, 0)),
        ],
        out_specs=[
            pl.BlockSpec((NC, BM, H1 // 2), lambda i: (0, i, 0)),
            _dinv_spec(),
        ],
        out_shape=[
            jax.ShapeDtypeStruct((NC, N_NODES, H1 // 2), jnp.float32),
            jax.ShapeDtypeStruct((N_NODES, 1), jnp.float32),
        ],
    )(x, W1, degp)

    p1 = _scatter_h1(srcT, dstT, h1p)

    h2p = pl.pallas_call(
        _mid_body,
        grid=(GRID,),
        in_specs=[
            pl.BlockSpec((NC, BM, H1 // 2), lambda i: (0, i, 0)),
            pl.BlockSpec((NC, BM, H1 // 2), lambda i: (0, i, 0)),
            _dinv_spec(),
            pl.BlockSpec((1, H1), lambda i: (0, 0)),
            pl.BlockSpec((H1, H2), lambda i: (0, 0)),
        ],
        out_specs=pl.BlockSpec((NC, BM, H2 // 2), lambda i: (0, i, 0)),
        out_shape=jax.ShapeDtypeStruct((NC, N_NODES, H2 // 2), jnp.float32),
    )(h1p, p1, dinv, b1.reshape(1, H1), W2)

    p2 = _scatter_h2(srcT, dstT, h2p)

    out = pl.pallas_call(
        _out_body,
        grid=(GRID,),
        in_specs=[
            pl.BlockSpec((NC, BM, H2 // 2), lambda i: (0, i, 0)),
            pl.BlockSpec((NC, BM, H2 // 2), lambda i: (0, i, 0)),
            _dinv_spec(),
            pl.BlockSpec((1, H2), lambda i: (0, 0)),
        ],
        out_specs=pl.BlockSpec((BM, H2), lambda i: (i, 0)),
        out_shape=jax.ShapeDtypeStruct((N_NODES, H2), jnp.float32),
    )(h2p, p2, dinv, b2.reshape(1, H2))

    return out
